# baseline (speedup 1.0000x reference)
import functools

import jax
import jax.numpy as jnp
from jax import lax
from jax.experimental import pallas as pl
from jax.experimental.pallas import tpu as pltpu
from jax.experimental.pallas import tpu_sc as plsc

EPS = 1e-05

S = 2048
D = 1024
E = 8
FF = 1024
ROWS_A = 256

NC, NS = 2, 16
M = 128
NT = 40
P = NT * M
TT = S // NS
GR = P // (NC * NS)
CT = S // (NC * NS)

_MESH = plsc.VectorSubcoreMesh(core_axis_name="c", subcore_axis_name="s",
                               num_cores=NC, num_subcores=NS)
_SC_PARAMS = pltpu.CompilerParams(needs_layout_passes=False)


def _rms(x, w):
    var = jnp.mean(x * x, axis=-1, keepdims=True)
    return x * jax.lax.rsqrt(var + EPS) * w


def _dot_t(a, b):
    return jax.lax.dot_general(a, b, (((1,), (1,)), ((), ())),
                               preferred_element_type=jnp.float32)


def _pre_moe_body(hs_ref, res_ref, hs_prev_ref, res_prev_ref, opw_ref,
                  ffw_ref, cin_ref, convt_ref, cout_ref, gw_ref, eb_ref,
                  res_out_ref, h2_ref, i1_ref, i2_ref, w0_ref, w1_ref,
                  cnt_ref, cacc_ref):
    i = pl.program_id(0)
    z = hs_ref[...] + res_ref[...]
    h = _rms(z, opw_ref[...])
    bcx = _dot_t(h, cin_ref[...])
    b = bcx[:, :D]
    c = bcx[:, D:2 * D]
    x = bcx[:, 2 * D:]
    bx = b * x
    zp = hs_prev_ref[ROWS_A - 2:, :] + res_prev_ref[ROWS_A - 2:, :]
    hp = _rms(zp, opw_ref[...])
    bxh = _dot_t(hp, cin_ref[:D, :]) * _dot_t(hp, cin_ref[2 * D:, :])
    bxh = jnp.where(i > 0, bxh, 0.0)
    bxp = jnp.concatenate([bxh, bx], axis=0)
    conv = (bxp[0:ROWS_A] * convt_ref[0:1, :]
            + bxp[1:ROWS_A + 1] * convt_ref[1:2, :]
            + bxp[2:ROWS_A + 2] * convt_ref[2:3, :])
    y = c * conv
    z2 = _dot_t(y, cout_ref[...]) + z
    res_out_ref[...] = z2
    h2 = _rms(z2, ffw_ref[...])
    h2_ref[...] = h2
    logits = _dot_t(h2, gw_ref[...])
    sc = jax.nn.sigmoid(logits)
    ch = sc + eb_ref[...]
    m1 = ch[:, 0:1]
    s1 = sc[:, 0:1]
    i1 = jnp.zeros((ROWS_A, 1), jnp.int32)
    for k in range(1, E):
        ck = ch[:, k:k + 1]
        upd = ck > m1
        i1 = jnp.where(upd, k, i1)
        s1 = jnp.where(upd, sc[:, k:k + 1], s1)
        m1 = jnp.where(upd, ck, m1)
    m2 = jnp.full((ROWS_A, 1), -jnp.inf, jnp.float32)
    s2 = jnp.zeros((ROWS_A, 1), jnp.float32)
    i2 = jnp.zeros((ROWS_A, 1), jnp.int32)
    for k in range(E):
        ck = ch[:, k:k + 1]
        upd = jnp.logical_and(i1 != k, ck > m2)
        i2 = jnp.where(upd, k, i2)
        s2 = jnp.where(upd, sc[:, k:k + 1], s2)
        m2 = jnp.where(upd, ck, m2)
    den = s1 + s2
    i1_ref[...] = i1
    i2_ref[...] = i2
    w0_ref[...] = s1 / den
    w1_ref[...] = s2 / den
    part = jnp.zeros((1, 16), jnp.int32)
    for e in range(E):
        ce = (jnp.sum((i1 == e).astype(jnp.int32))
              + jnp.sum((i2 == e).astype(jnp.int32)))
        onehot = (lax.broadcasted_iota(jnp.int32, (1, 16), 1) == e)
        part = part + jnp.where(onehot, ce, 0)

    @pl.when(i == 0)
    def _():
        cacc_ref[...] = part

    @pl.when(i > 0)
    def _():
        cacc_ref[...] += part

    @pl.when(i == S // ROWS_A - 1)
    def _():
        cnt_ref[...] = cacc_ref[...]


def _pre_moe(hs, res, opw, ffw, cin, convt, cout, gw, eb):
    n = S // ROWS_A
    row_blk = pl.BlockSpec((ROWS_A, D), lambda i: (i, 0))
    prev_blk = pl.BlockSpec((ROWS_A, D), lambda i: (jnp.maximum(i - 1, 0), 0))
    col_blk = pl.BlockSpec((ROWS_A, 1), lambda i: (i, 0))
    const2 = lambda shape: pl.BlockSpec(shape, lambda i: (0, 0))
    out_shapes = [
        jax.ShapeDtypeStruct((S, D), jnp.float32),
        jax.ShapeDtypeStruct((S, D), jnp.float32),
        jax.ShapeDtypeStruct((S, 1), jnp.int32),
        jax.ShapeDtypeStruct((S, 1), jnp.int32),
        jax.ShapeDtypeStruct((S, 1), jnp.float32),
        jax.ShapeDtypeStruct((S, 1), jnp.float32),
        jax.ShapeDtypeStruct((1, 16), jnp.int32),
    ]
    return pl.pallas_call(
        _pre_moe_body,
        grid=(n,),
        in_specs=[row_blk, row_blk, prev_blk, prev_blk,
                  const2((1, D)), const2((1, D)), const2((3 * D, D)),
                  const2((3, D)), const2((D, D)), const2((E, D)),
                  const2((1, E))],
        out_specs=[row_blk, row_blk, col_blk, col_blk, col_blk, col_blk,
                   pl.BlockSpec((1, 16), lambda i: (0, 0))],
        out_shape=out_shapes,
        scratch_shapes=[pltpu.VMEM((1, 16), jnp.int32)],
    )(hs, res, hs, res, opw.reshape(1, D), ffw.reshape(1, D), cin, convt,
      cout, gw, eb.reshape(1, E))


_NR = S // 128


def _routing_body(i1_hbm, i2_hbm, cnt_hbm,
                  slot_tok_hbm, pos0_hbm, pos1_hbm,
                  te_hbm, act_hbm,
                  iv1_v, iv2_v, p1_v, p2_v, z_v,
                  run_v, ue_v, te_v, act_v):
    c = lax.axis_index("c")
    s = lax.axis_index("s")

    @pl.when(jnp.logical_and(c == 0, s == 0))
    def _():
        lanes = lax.iota(jnp.int32, 16)
        for j in range(0, P, 16):
            z_v[pl.ds(j, 16)] = jnp.full((16,), j & (S - 1), jnp.int32) + lanes
        pltpu.sync_copy(i1_hbm, iv1_v)
        pltpu.sync_copy(i2_hbm, iv2_v)
        pltpu.sync_copy(cnt_hbm, run_v)
        tot = run_v[...]
        padded = ((tot + (M - 1)) >> 7) << 7
        apo_inc = plsc.cumsum(padded)
        apo_exc = apo_inc - padded
        run_v[...] = apo_exc

        def pbody(r, z):
            for gg in range(128 // 16):
                sl = pl.ds(gg * 16, 16)
                tokv = r * 128 + gg * 16 + lanes
                for iref, pref in ((iv1_v, p1_v), (iv2_v, p2_v)):
                    vec = iref[r, sl]
                    basev = plsc.load_gather(run_v, [vec])
                    within = jnp.zeros((16,), jnp.int32)
                    cntvec = jnp.zeros((16,), jnp.int32)
                    for e in range(E):
                        m = vec == e
                        mi = m.astype(jnp.int32)
                        cm = plsc.cumsum(mi)
                        within = jnp.where(m, cm - 1, within)
                        cntvec = cntvec + jnp.where(lanes == e, jnp.sum(mi), 0)
                    posv = basev + within
                    pref[r, sl] = posv
                    run_v[...] = run_v[...] + cntvec
                    plsc.store_scatter(z_v, [posv], tokv)
            return z

        lax.fori_loop(0, _NR, pbody, 0)
        pltpu.sync_copy(p1_v, pos0_hbm)
        pltpu.sync_copy(p2_v, pos1_hbm)
        pltpu.sync_copy(z_v, slot_tok_hbm)

        ue_v[...] = apo_exc + tot
        for half in range(3):
            tb = (lax.iota(jnp.int32, 16) + half * 16) * M
            te = jnp.zeros((16,), jnp.int32)
            for e in range(E):
                ae = jnp.sum(jnp.where(lanes == e, apo_inc, 0))
                te = te + (tb >= ae).astype(jnp.int32)
            te = jnp.minimum(te, E - 1)
            ueg = plsc.load_gather(ue_v, [te])
            act = (tb < ueg).astype(jnp.int32)
            te_v[pl.ds(half * 16, 16)] = te
            act_v[pl.ds(half * 16, 16)] = act
        pltpu.sync_copy(te_v, te_hbm)
        pltpu.sync_copy(act_v, act_hbm)


def _routing(i1f, i2f, cnt):
    out_type = [
        jax.ShapeDtypeStruct((P,), jnp.int32),
        jax.ShapeDtypeStruct((_NR, 128), jnp.int32),
        jax.ShapeDtypeStruct((_NR, 128), jnp.int32),
        jax.ShapeDtypeStruct((48,), jnp.int32),
        jax.ShapeDtypeStruct((48,), jnp.int32),
    ]
    scratch = [
        pltpu.VMEM((_NR, 128), jnp.int32),
        pltpu.VMEM((_NR, 128), jnp.int32),
        pltpu.VMEM((_NR, 128), jnp.int32),
        pltpu.VMEM((_NR, 128), jnp.int32),
        pltpu.VMEM((P,), jnp.int32),
        pltpu.VMEM((16,), jnp.int32),
        pltpu.VMEM((16,), jnp.int32),
        pltpu.VMEM((48,), jnp.int32),
        pltpu.VMEM((48,), jnp.int32),
    ]
    fn = pl.kernel(_routing_body, out_type=out_type, mesh=_MESH,
                   compiler_params=_SC_PARAMS, scratch_types=scratch)
    return fn(i1f, i2f, cnt)


GCH = 32
NCHG = GR // GCH


def _gather_body(tok_hbm, h2_hbm, xs_hbm, idx_v, buf0, buf1, sem0, sem1):
    c = lax.axis_index("c")
    s = lax.axis_index("s")
    wid = s * NC + c
    base = wid * GR
    pltpu.sync_copy(tok_hbm.at[pl.ds(base, GR)], idx_v)
    bufs = (buf0, buf1)
    sems = (sem0, sem1)
    cps = []
    for k in range(NCHG):
        cps.append(pltpu.async_copy(
            h2_hbm.at[idx_v.at[pl.ds(k * GCH, GCH)]], bufs[k % 2],
            sems[k % 2]))
        if k >= 1:
            cps[k - 1].wait()
            pltpu.sync_copy(bufs[(k - 1) % 2],
                            xs_hbm.at[pl.ds(base + (k - 1) * GCH, GCH)])
    cps[NCHG - 1].wait()
    pltpu.sync_copy(bufs[(NCHG - 1) % 2],
                    xs_hbm.at[pl.ds(base + (NCHG - 1) * GCH, GCH)])


def _gather(slot_tok, h2):
    fn = pl.kernel(
        _gather_body,
        out_type=jax.ShapeDtypeStruct((P, D), jnp.float32),
        mesh=_MESH,
        compiler_params=_SC_PARAMS,
        scratch_types=[
            pltpu.VMEM((GR,), jnp.int32),
            pltpu.VMEM((GCH, D), jnp.float32),
            pltpu.VMEM((GCH, D), jnp.float32),
            pltpu.SemaphoreType.DMA,
            pltpu.SemaphoreType.DMA,
        ],
    )
    return fn(slot_tok, h2)


def _moe_body(te_ref, act_ref, xs_ref, w1_ref, w2_ref, ys_ref):
    i = pl.program_id(0)

    @pl.when(act_ref[i] == 1)
    def _():
        x = xs_ref[...]
        gu = _dot_t(x, w1_ref[0])
        g = gu[:, :FF]
        u = gu[:, FF:]
        a = g * jax.nn.sigmoid(g) * u
        ys_ref[...] = _dot_t(a, w2_ref[0])


def _moe_matmul(te, act, xs, W1, W2):
    grid_spec = pltpu.PrefetchScalarGridSpec(
        num_scalar_prefetch=2,
        grid=(NT,),
        in_specs=[
            pl.BlockSpec((M, D), lambda i, te, act: (i, 0)),
            pl.BlockSpec((1, 2 * FF, D), lambda i, te, act: (te[i], 0, 0)),
            pl.BlockSpec((1, D, FF), lambda i, te, act: (te[i], 0, 0)),
        ],
        out_specs=pl.BlockSpec((M, D), lambda i, te, act: (i, 0)),
    )
    return pl.pallas_call(
        _moe_body,
        grid_spec=grid_spec,
        out_shape=jax.ShapeDtypeStruct((P, D), jnp.float32),
    )(te, act, xs, W1, W2)


CCH = 32
NCHC = CT // CCH


def _combine_gather_body(pos0_hbm, pos1_hbm, ys_hbm, g0_hbm, g1_hbm,
                         i0_v, i1_v, b0_v, b1_v, sem0, sem1):
    c = lax.axis_index("c")
    s = lax.axis_index("s")
    wid = s * NC + c
    base = wid * CT
    pltpu.sync_copy(pos0_hbm.at[pl.ds(base, CT)], i0_v)
    pltpu.sync_copy(pos1_hbm.at[pl.ds(base, CT)], i1_v)
    bufs = (b0_v, b1_v)
    sems = (sem0, sem1)
    plan = []
    for k in range(NCHC):
        plan.append((i0_v, g0_hbm, k))
        plan.append((i1_v, g1_hbm, k))
    cps = []
    for j, (iv, dst, k) in enumerate(plan):
        cps.append(pltpu.async_copy(
            ys_hbm.at[iv.at[pl.ds(k * CCH, CCH)]], bufs[j % 2], sems[j % 2]))
        if j >= 1:
            piv, pdst, pk = plan[j - 1]
            cps[j - 1].wait()
            pltpu.sync_copy(bufs[(j - 1) % 2],
                            pdst.at[pl.ds(base + pk * CCH, CCH)])
    j = len(plan) - 1
    piv, pdst, pk = plan[j]
    cps[j].wait()
    pltpu.sync_copy(bufs[j % 2], pdst.at[pl.ds(base + pk * CCH, CCH)])


def _combine_gather(pos0, pos1, ys):
    fn = pl.kernel(
        _combine_gather_body,
        out_type=[jax.ShapeDtypeStruct((S, D), jnp.float32),
                  jax.ShapeDtypeStruct((S, D), jnp.float32)],
        mesh=_MESH,
        compiler_params=_SC_PARAMS,
        scratch_types=[
            pltpu.VMEM((CT,), jnp.int32),
            pltpu.VMEM((CT,), jnp.int32),
            pltpu.VMEM((CCH, D), jnp.float32),
            pltpu.VMEM((CCH, D), jnp.float32),
            pltpu.SemaphoreType.DMA,
            pltpu.SemaphoreType.DMA,
        ],
    )
    return fn(pos0, pos1, ys)


def _combine_add_body(g0_ref, g1_ref, w0_ref, w1_ref, out_ref):
    out_ref[...] = g0_ref[...] * w0_ref[...] + g1_ref[...] * w1_ref[...]


def _combine_add(g0, g1, w0, w1):
    rows = 512
    blk = pl.BlockSpec((rows, D), lambda i: (i, 0))
    cblk = pl.BlockSpec((rows, 1), lambda i: (i, 0))
    return pl.pallas_call(
        _combine_add_body,
        grid=(S // rows,),
        in_specs=[blk, blk, cblk, cblk],
        out_specs=blk,
        out_shape=jax.ShapeDtypeStruct((S, D), jnp.float32),
    )(g0, g1, w0, w1)


def kernel(hidden_states, residual, op_norm_w, ffn_norm_w, conv_in_W,
           conv_W, conv_out_W, gate_W, e_bias, W1, W2):
    B = hidden_states.shape[0]
    hs = hidden_states.reshape(S, D)
    res = residual.reshape(S, D)
    convt = conv_W.T
    res_out, h2, i1, i2, w0, w1, cnt = _pre_moe(
        hs, res, op_norm_w, ffn_norm_w, conv_in_W, convt, conv_out_W,
        gate_W, e_bias)
    slot_tok, pos0, pos1, te, act = _routing(
        i1.reshape(_NR, 128), i2.reshape(_NR, 128), cnt.reshape(16))
    xs = _gather(slot_tok, h2)
    ys = _moe_matmul(te, act, xs, W1, W2)
    g0, g1 = _combine_gather(pos0.reshape(S), pos1.reshape(S), ys)
    out = _combine_add(g0, g1, w0, w1)
    return out.reshape(B, S, D), res_out.reshape(B, S, D)


# speedup vs baseline: 1.1626x; 1.1626x over previous
import functools

import jax
import jax.numpy as jnp
from jax import lax
from jax.experimental import pallas as pl
from jax.experimental.pallas import tpu as pltpu
from jax.experimental.pallas import tpu_sc as plsc

EPS = 1e-05

S = 2048
D = 1024
E = 8
FF = 1024
ROWS_A = 256

NC, NS = 2, 16
M = 256
NT = 24
P = NT * M
TT = S // NS
GR = P // (NC * NS)
CT = S // (NC * NS)

_MESH = plsc.VectorSubcoreMesh(core_axis_name="c", subcore_axis_name="s",
                               num_cores=NC, num_subcores=NS)
_SC_PARAMS = pltpu.CompilerParams(needs_layout_passes=False)


def _rms(x, w):
    var = jnp.mean(x * x, axis=-1, keepdims=True)
    return x * jax.lax.rsqrt(var + EPS) * w


def _dot_t(a, b):
    return jax.lax.dot_general(a, b, (((1,), (1,)), ((), ())),
                               preferred_element_type=jnp.float32)


def _pre_moe_body(hs_ref, res_ref, hs_prev_ref, res_prev_ref, opw_ref,
                  ffw_ref, cin_ref, convt_ref, cout_ref, gw_ref, eb_ref,
                  res_out_ref, h2_ref, i1_ref, i2_ref, w0_ref, w1_ref,
                  cnt_ref, cacc_ref):
    i = pl.program_id(0)
    z = hs_ref[...] + res_ref[...]
    h = _rms(z, opw_ref[...])
    bcx = _dot_t(h, cin_ref[...])
    b = bcx[:, :D]
    c = bcx[:, D:2 * D]
    x = bcx[:, 2 * D:]
    bx = b * x
    zp = hs_prev_ref[ROWS_A - 2:, :] + res_prev_ref[ROWS_A - 2:, :]
    hp = _rms(zp, opw_ref[...])
    bxh = _dot_t(hp, cin_ref[:D, :]) * _dot_t(hp, cin_ref[2 * D:, :])
    bxh = jnp.where(i > 0, bxh, 0.0)
    bxp = jnp.concatenate([bxh, bx], axis=0)
    conv = (bxp[0:ROWS_A] * convt_ref[0:1, :]
            + bxp[1:ROWS_A + 1] * convt_ref[1:2, :]
            + bxp[2:ROWS_A + 2] * convt_ref[2:3, :])
    y = c * conv
    z2 = _dot_t(y, cout_ref[...]) + z
    res_out_ref[...] = z2
    h2 = _rms(z2, ffw_ref[...])
    h2_ref[...] = h2
    logits = _dot_t(h2, gw_ref[...])
    sc = jax.nn.sigmoid(logits)
    ch = sc + eb_ref[...]
    m1 = ch[:, 0:1]
    s1 = sc[:, 0:1]
    i1 = jnp.zeros((ROWS_A, 1), jnp.int32)
    for k in range(1, E):
        ck = ch[:, k:k + 1]
        upd = ck > m1
        i1 = jnp.where(upd, k, i1)
        s1 = jnp.where(upd, sc[:, k:k + 1], s1)
        m1 = jnp.where(upd, ck, m1)
    m2 = jnp.full((ROWS_A, 1), -jnp.inf, jnp.float32)
    s2 = jnp.zeros((ROWS_A, 1), jnp.float32)
    i2 = jnp.zeros((ROWS_A, 1), jnp.int32)
    for k in range(E):
        ck = ch[:, k:k + 1]
        upd = jnp.logical_and(i1 != k, ck > m2)
        i2 = jnp.where(upd, k, i2)
        s2 = jnp.where(upd, sc[:, k:k + 1], s2)
        m2 = jnp.where(upd, ck, m2)
    den = s1 + s2
    i1_ref[...] = i1
    i2_ref[...] = i2
    w0_ref[...] = s1 / den
    w1_ref[...] = s2 / den
    part = jnp.zeros((1, 16), jnp.int32)
    for e in range(E):
        ce = (jnp.sum((i1 == e).astype(jnp.int32))
              + jnp.sum((i2 == e).astype(jnp.int32)))
        onehot = (lax.broadcasted_iota(jnp.int32, (1, 16), 1) == e)
        part = part + jnp.where(onehot, ce, 0)

    @pl.when(i == 0)
    def _():
        cacc_ref[...] = part

    @pl.when(i > 0)
    def _():
        cacc_ref[...] += part

    @pl.when(i == S // ROWS_A - 1)
    def _():
        cnt_ref[...] = cacc_ref[...]


def _pre_moe(hs, res, opw, ffw, cin, convt, cout, gw, eb):
    n = S // ROWS_A
    row_blk = pl.BlockSpec((ROWS_A, D), lambda i: (i, 0))
    prev_blk = pl.BlockSpec((ROWS_A, D), lambda i: (jnp.maximum(i - 1, 0), 0))
    col_blk = pl.BlockSpec((ROWS_A, 1), lambda i: (i, 0))
    const2 = lambda shape: pl.BlockSpec(shape, lambda i: (0, 0))
    out_shapes = [
        jax.ShapeDtypeStruct((S, D), jnp.float32),
        jax.ShapeDtypeStruct((S, D), jnp.float32),
        jax.ShapeDtypeStruct((S, 1), jnp.int32),
        jax.ShapeDtypeStruct((S, 1), jnp.int32),
        jax.ShapeDtypeStruct((S, 1), jnp.float32),
        jax.ShapeDtypeStruct((S, 1), jnp.float32),
        jax.ShapeDtypeStruct((1, 16), jnp.int32),
    ]
    return pl.pallas_call(
        _pre_moe_body,
        grid=(n,),
        in_specs=[row_blk, row_blk, prev_blk, prev_blk,
                  const2((1, D)), const2((1, D)), const2((3 * D, D)),
                  const2((3, D)), const2((D, D)), const2((E, D)),
                  const2((1, E))],
        out_specs=[row_blk, row_blk, col_blk, col_blk, col_blk, col_blk,
                   pl.BlockSpec((1, 16), lambda i: (0, 0))],
        out_shape=out_shapes,
        scratch_shapes=[pltpu.VMEM((1, 16), jnp.int32)],
    )(hs, res, hs, res, opw.reshape(1, D), ffw.reshape(1, D), cin, convt,
      cout, gw, eb.reshape(1, E))


_NR = S // 128


def _routing_body(i1_hbm, i2_hbm, cnt_hbm,
                  slot_tok_hbm, pos0_hbm, pos1_hbm,
                  te_hbm, act_hbm,
                  iv1_v, iv2_v, p1_v, p2_v, z_v,
                  run_v, ue_v, te_v, act_v):
    c = lax.axis_index("c")
    s = lax.axis_index("s")

    @pl.when(jnp.logical_and(c == 0, s == 0))
    def _():
        lanes = lax.iota(jnp.int32, 16)
        for j in range(0, P, 16):
            z_v[pl.ds(j, 16)] = jnp.full((16,), j & (S - 1), jnp.int32) + lanes
        pltpu.sync_copy(i1_hbm, iv1_v)
        pltpu.sync_copy(i2_hbm, iv2_v)
        pltpu.sync_copy(cnt_hbm, run_v)
        tot = run_v[...]
        padded = ((tot + (M - 1)) >> 8) << 8
        apo_inc = plsc.cumsum(padded)
        apo_exc = apo_inc - padded
        run_v[...] = apo_exc

        def pbody(r, z):
            for gg in range(128 // 16):
                sl = pl.ds(gg * 16, 16)
                tokv = r * 128 + gg * 16 + lanes
                for iref, pref in ((iv1_v, p1_v), (iv2_v, p2_v)):
                    vec = iref[r, sl]
                    basev = plsc.load_gather(run_v, [vec])
                    within = jnp.zeros((16,), jnp.int32)
                    cntvec = jnp.zeros((16,), jnp.int32)
                    for e in range(E):
                        m = vec == e
                        mi = m.astype(jnp.int32)
                        cm = plsc.cumsum(mi)
                        within = jnp.where(m, cm - 1, within)
                        cntvec = cntvec + jnp.where(lanes == e, jnp.sum(mi), 0)
                    posv = basev + within
                    pref[r, sl] = posv
                    run_v[...] = run_v[...] + cntvec
                    plsc.store_scatter(z_v, [posv], tokv)
            return z

        lax.fori_loop(0, _NR, pbody, 0)
        pltpu.sync_copy(p1_v, pos0_hbm)
        pltpu.sync_copy(p2_v, pos1_hbm)
        pltpu.sync_copy(z_v, slot_tok_hbm)

        ue_v[...] = apo_exc + tot
        for half in range(2):
            tb = (lax.iota(jnp.int32, 16) + half * 16) * M
            te = jnp.zeros((16,), jnp.int32)
            for e in range(E):
                ae = jnp.sum(jnp.where(lanes == e, apo_inc, 0))
                te = te + (tb >= ae).astype(jnp.int32)
            te = jnp.minimum(te, E - 1)
            ueg = plsc.load_gather(ue_v, [te])
            act = (tb < ueg).astype(jnp.int32)
            te_v[pl.ds(half * 16, 16)] = te
            act_v[pl.ds(half * 16, 16)] = act
        pltpu.sync_copy(te_v, te_hbm)
        pltpu.sync_copy(act_v, act_hbm)


def _routing(i1f, i2f, cnt):
    out_type = [
        jax.ShapeDtypeStruct((P,), jnp.int32),
        jax.ShapeDtypeStruct((_NR, 128), jnp.int32),
        jax.ShapeDtypeStruct((_NR, 128), jnp.int32),
        jax.ShapeDtypeStruct((32,), jnp.int32),
        jax.ShapeDtypeStruct((32,), jnp.int32),
    ]
    scratch = [
        pltpu.VMEM((_NR, 128), jnp.int32),
        pltpu.VMEM((_NR, 128), jnp.int32),
        pltpu.VMEM((_NR, 128), jnp.int32),
        pltpu.VMEM((_NR, 128), jnp.int32),
        pltpu.VMEM((P,), jnp.int32),
        pltpu.VMEM((16,), jnp.int32),
        pltpu.VMEM((16,), jnp.int32),
        pltpu.VMEM((32,), jnp.int32),
        pltpu.VMEM((32,), jnp.int32),
    ]
    fn = pl.kernel(_routing_body, out_type=out_type, mesh=_MESH,
                   compiler_params=_SC_PARAMS, scratch_types=scratch)
    return fn(i1f, i2f, cnt)


GCH = 48
NCHG = GR // GCH


def _gather_body(tok_hbm, h2_hbm, xs_hbm, idx_v, buf0, buf1, sem0, sem1):
    c = lax.axis_index("c")
    s = lax.axis_index("s")
    wid = s * NC + c
    base = wid * GR
    pltpu.sync_copy(tok_hbm.at[pl.ds(base, GR)], idx_v)
    bufs = (buf0, buf1)
    sems = (sem0, sem1)
    cps = []
    for k in range(NCHG):
        cps.append(pltpu.async_copy(
            h2_hbm.at[idx_v.at[pl.ds(k * GCH, GCH)]], bufs[k % 2],
            sems[k % 2]))
        if k >= 1:
            cps[k - 1].wait()
            pltpu.sync_copy(bufs[(k - 1) % 2],
                            xs_hbm.at[pl.ds(base + (k - 1) * GCH, GCH)])
    cps[NCHG - 1].wait()
    pltpu.sync_copy(bufs[(NCHG - 1) % 2],
                    xs_hbm.at[pl.ds(base + (NCHG - 1) * GCH, GCH)])


def _gather(slot_tok, h2):
    fn = pl.kernel(
        _gather_body,
        out_type=jax.ShapeDtypeStruct((P, D), jnp.float32),
        mesh=_MESH,
        compiler_params=_SC_PARAMS,
        scratch_types=[
            pltpu.VMEM((GR,), jnp.int32),
            pltpu.VMEM((GCH, D), jnp.float32),
            pltpu.VMEM((GCH, D), jnp.float32),
            pltpu.SemaphoreType.DMA,
            pltpu.SemaphoreType.DMA,
        ],
    )
    return fn(slot_tok, h2)


def _moe_body(te_ref, act_ref, xs_ref, w1_ref, w2_ref, ys_ref):
    i = pl.program_id(0)

    @pl.when(act_ref[i] == 1)
    def _():
        x = xs_ref[...]
        gu = _dot_t(x, w1_ref[0])
        g = gu[:, :FF]
        u = gu[:, FF:]
        a = g * jax.nn.sigmoid(g) * u
        ys_ref[...] = _dot_t(a, w2_ref[0])


def _moe_matmul(te, act, xs, W1, W2):
    grid_spec = pltpu.PrefetchScalarGridSpec(
        num_scalar_prefetch=2,
        grid=(NT,),
        in_specs=[
            pl.BlockSpec((M, D), lambda i, te, act: (i, 0)),
            pl.BlockSpec((1, 2 * FF, D), lambda i, te, act: (te[i], 0, 0)),
            pl.BlockSpec((1, D, FF), lambda i, te, act: (te[i], 0, 0)),
        ],
        out_specs=pl.BlockSpec((M, D), lambda i, te, act: (i, 0)),
    )
    return pl.pallas_call(
        _moe_body,
        grid_spec=grid_spec,
        out_shape=jax.ShapeDtypeStruct((P, D), jnp.float32),
    )(te, act, xs, W1, W2)


CCH = 32
NCHC = CT // CCH


def _combine_gather_body(pos0_hbm, pos1_hbm, ys_hbm, g0_hbm, g1_hbm,
                         i0_v, i1_v, b0_v, b1_v, sem0, sem1):
    c = lax.axis_index("c")
    s = lax.axis_index("s")
    wid = s * NC + c
    base = wid * CT
    pltpu.sync_copy(pos0_hbm.at[pl.ds(base, CT)], i0_v)
    pltpu.sync_copy(pos1_hbm.at[pl.ds(base, CT)], i1_v)
    bufs = (b0_v, b1_v)
    sems = (sem0, sem1)
    plan = []
    for k in range(NCHC):
        plan.append((i0_v, g0_hbm, k))
        plan.append((i1_v, g1_hbm, k))
    cps = []
    for j, (iv, dst, k) in enumerate(plan):
        cps.append(pltpu.async_copy(
            ys_hbm.at[iv.at[pl.ds(k * CCH, CCH)]], bufs[j % 2], sems[j % 2]))
        if j >= 1:
            piv, pdst, pk = plan[j - 1]
            cps[j - 1].wait()
            pltpu.sync_copy(bufs[(j - 1) % 2],
                            pdst.at[pl.ds(base + pk * CCH, CCH)])
    j = len(plan) - 1
    piv, pdst, pk = plan[j]
    cps[j].wait()
    pltpu.sync_copy(bufs[j % 2], pdst.at[pl.ds(base + pk * CCH, CCH)])


def _combine_gather(pos0, pos1, ys):
    fn = pl.kernel(
        _combine_gather_body,
        out_type=[jax.ShapeDtypeStruct((S, D), jnp.float32),
                  jax.ShapeDtypeStruct((S, D), jnp.float32)],
        mesh=_MESH,
        compiler_params=_SC_PARAMS,
        scratch_types=[
            pltpu.VMEM((CT,), jnp.int32),
            pltpu.VMEM((CT,), jnp.int32),
            pltpu.VMEM((CCH, D), jnp.float32),
            pltpu.VMEM((CCH, D), jnp.float32),
            pltpu.SemaphoreType.DMA,
            pltpu.SemaphoreType.DMA,
        ],
    )
    return fn(pos0, pos1, ys)


def _combine_add_body(g0_ref, g1_ref, w0_ref, w1_ref, out_ref):
    out_ref[...] = g0_ref[...] * w0_ref[...] + g1_ref[...] * w1_ref[...]


def _combine_add(g0, g1, w0, w1):
    rows = 512
    blk = pl.BlockSpec((rows, D), lambda i: (i, 0))
    cblk = pl.BlockSpec((rows, 1), lambda i: (i, 0))
    return pl.pallas_call(
        _combine_add_body,
        grid=(S // rows,),
        in_specs=[blk, blk, cblk, cblk],
        out_specs=blk,
        out_shape=jax.ShapeDtypeStruct((S, D), jnp.float32),
    )(g0, g1, w0, w1)


def kernel(hidden_states, residual, op_norm_w, ffn_norm_w, conv_in_W,
           conv_W, conv_out_W, gate_W, e_bias, W1, W2):
    B = hidden_states.shape[0]
    hs = hidden_states.reshape(S, D)
    res = residual.reshape(S, D)
    convt = conv_W.T
    res_out, h2, i1, i2, w0, w1, cnt = _pre_moe(
        hs, res, op_norm_w, ffn_norm_w, conv_in_W, convt, conv_out_W,
        gate_W, e_bias)
    slot_tok, pos0, pos1, te, act = _routing(
        i1.reshape(_NR, 128), i2.reshape(_NR, 128), cnt.reshape(16))
    xs = _gather(slot_tok, h2)
    ys = _moe_matmul(te, act, xs, W1, W2)
    g0, g1 = _combine_gather(pos0.reshape(S), pos1.reshape(S), ys)
    out = _combine_add(g0, g1, w0, w1)
    return out.reshape(B, S, D), res_out.reshape(B, S, D)


# speedup vs baseline: 1.1637x; 1.0009x over previous
import jax
import jax.numpy as jnp
from jax import lax
from jax.experimental import pallas as pl
from jax.experimental.pallas import tpu as pltpu
from jax.experimental.pallas import tpu_sc as plsc

EPS = 1e-05

S = 2048
D = 1024
E = 8
FF = 1024
ROWS_A = 256

NC, NS = 2, 16
M = 256
NT = 24
P = NT * M
TT = S // NS
GR = P // (NC * NS)
CT = S // (NC * NS)

_MESH = plsc.VectorSubcoreMesh(core_axis_name="c", subcore_axis_name="s",
                               num_cores=NC, num_subcores=NS)
_SC_PARAMS = pltpu.CompilerParams(needs_layout_passes=False)


def _rms(x, w):
    var = jnp.mean(x * x, axis=-1, keepdims=True)
    return x * jax.lax.rsqrt(var + EPS) * w


def _dot_t(a, b):
    return jax.lax.dot_general(a, b, (((1,), (1,)), ((), ())),
                               preferred_element_type=jnp.float32)


def _pre_moe_body(hs_ref, res_ref, hs_prev_ref, res_prev_ref, opw_ref,
                  ffw_ref, cin_ref, convt_ref, cout_ref, gw_ref, eb_ref,
                  res_out_ref, h2_ref, i1_ref, i2_ref, w0_ref, w1_ref,
                  cnt_ref, cacc_ref):
    i = pl.program_id(0)
    z = hs_ref[...] + res_ref[...]
    h = _rms(z, opw_ref[...])
    bcx = _dot_t(h, cin_ref[...])
    b = bcx[:, :D]
    c = bcx[:, D:2 * D]
    x = bcx[:, 2 * D:]
    bx = b * x
    zp = hs_prev_ref[ROWS_A - 2:, :] + res_prev_ref[ROWS_A - 2:, :]
    hp = _rms(zp, opw_ref[...])
    bxh = _dot_t(hp, cin_ref[:D, :]) * _dot_t(hp, cin_ref[2 * D:, :])
    bxh = jnp.where(i > 0, bxh, 0.0)
    bxp = jnp.concatenate([bxh, bx], axis=0)
    conv = (bxp[0:ROWS_A] * convt_ref[0:1, :]
            + bxp[1:ROWS_A + 1] * convt_ref[1:2, :]
            + bxp[2:ROWS_A + 2] * convt_ref[2:3, :])
    y = c * conv
    z2 = _dot_t(y, cout_ref[...]) + z
    res_out_ref[...] = z2
    h2 = _rms(z2, ffw_ref[...])
    h2_ref[...] = h2
    logits = _dot_t(h2, gw_ref[...])
    sc = jax.nn.sigmoid(logits)
    ch = sc + eb_ref[...]
    m1 = ch[:, 0:1]
    s1 = sc[:, 0:1]
    i1 = jnp.zeros((ROWS_A, 1), jnp.int32)
    for k in range(1, E):
        ck = ch[:, k:k + 1]
        upd = ck > m1
        i1 = jnp.where(upd, k, i1)
        s1 = jnp.where(upd, sc[:, k:k + 1], s1)
        m1 = jnp.where(upd, ck, m1)
    m2 = jnp.full((ROWS_A, 1), -jnp.inf, jnp.float32)
    s2 = jnp.zeros((ROWS_A, 1), jnp.float32)
    i2 = jnp.zeros((ROWS_A, 1), jnp.int32)
    for k in range(E):
        ck = ch[:, k:k + 1]
        upd = jnp.logical_and(i1 != k, ck > m2)
        i2 = jnp.where(upd, k, i2)
        s2 = jnp.where(upd, sc[:, k:k + 1], s2)
        m2 = jnp.where(upd, ck, m2)
    den = s1 + s2
    i1_ref[...] = i1
    i2_ref[...] = i2
    w0_ref[...] = s1 / den
    w1_ref[...] = s2 / den
    part = jnp.zeros((1, 16), jnp.int32)
    for e in range(E):
        ce = (jnp.sum((i1 == e).astype(jnp.int32))
              + jnp.sum((i2 == e).astype(jnp.int32)))
        onehot = (lax.broadcasted_iota(jnp.int32, (1, 16), 1) == e)
        part = part + jnp.where(onehot, ce, 0)

    @pl.when(i == 0)
    def _():
        cacc_ref[...] = part

    @pl.when(i > 0)
    def _():
        cacc_ref[...] += part

    @pl.when(i == S // ROWS_A - 1)
    def _():
        cnt_ref[...] = cacc_ref[...]


def _pre_moe(hs, res, opw, ffw, cin, convt, cout, gw, eb):
    n = S // ROWS_A
    row_blk = pl.BlockSpec((ROWS_A, D), lambda i: (i, 0))
    prev_blk = pl.BlockSpec((ROWS_A, D), lambda i: (jnp.maximum(i - 1, 0), 0))
    col_blk = pl.BlockSpec((ROWS_A, 1), lambda i: (i, 0))
    const2 = lambda shape: pl.BlockSpec(shape, lambda i: (0, 0))
    out_shapes = [
        jax.ShapeDtypeStruct((S, D), jnp.float32),
        jax.ShapeDtypeStruct((S, D), jnp.float32),
        jax.ShapeDtypeStruct((S, 1), jnp.int32),
        jax.ShapeDtypeStruct((S, 1), jnp.int32),
        jax.ShapeDtypeStruct((S, 1), jnp.float32),
        jax.ShapeDtypeStruct((S, 1), jnp.float32),
        jax.ShapeDtypeStruct((1, 16), jnp.int32),
    ]
    return pl.pallas_call(
        _pre_moe_body,
        grid=(n,),
        in_specs=[row_blk, row_blk, prev_blk, prev_blk,
                  const2((1, D)), const2((1, D)), const2((3 * D, D)),
                  const2((3, D)), const2((D, D)), const2((E, D)),
                  const2((1, E))],
        out_specs=[row_blk, row_blk, col_blk, col_blk, col_blk, col_blk,
                   pl.BlockSpec((1, 16), lambda i: (0, 0))],
        out_shape=out_shapes,
        scratch_shapes=[pltpu.VMEM((1, 16), jnp.int32)],
    )(hs, res, hs, res, opw.reshape(1, D), ffw.reshape(1, D), cin, convt,
      cout, gw, eb.reshape(1, E))


_NR = S // 128


def _routing_body(i1_hbm, i2_hbm, cnt_hbm,
                  slot_tok_hbm, pos0_hbm, pos1_hbm,
                  te_hbm, act_hbm,
                  iv1_v, iv2_v, p1_v, p2_v, z_v,
                  run_v, ue_v, te_v, act_v):
    c = lax.axis_index("c")
    s = lax.axis_index("s")

    @pl.when(jnp.logical_and(c == 0, s == 0))
    def _():
        lanes = lax.iota(jnp.int32, 16)
        for j in range(0, P, 16):
            z_v[pl.ds(j, 16)] = jnp.full((16,), j & (S - 1), jnp.int32) + lanes
        pltpu.sync_copy(i1_hbm, iv1_v)
        pltpu.sync_copy(i2_hbm, iv2_v)
        pltpu.sync_copy(cnt_hbm, run_v)
        tot = run_v[...]
        padded = ((tot + (M - 1)) >> 8) << 8
        apo_inc = plsc.cumsum(padded)
        apo_exc = apo_inc - padded
        run_v[...] = apo_exc

        def pbody(r, z):
            for gg in range(128 // 16):
                sl = pl.ds(gg * 16, 16)
                tokv = r * 128 + gg * 16 + lanes
                for iref, pref in ((iv1_v, p1_v), (iv2_v, p2_v)):
                    vec = iref[r, sl]
                    basev = plsc.load_gather(run_v, [vec])
                    within = jnp.zeros((16,), jnp.int32)
                    cntvec = jnp.zeros((16,), jnp.int32)
                    for e in range(E):
                        m = vec == e
                        mi = m.astype(jnp.int32)
                        cm = plsc.cumsum(mi)
                        within = jnp.where(m, cm - 1, within)
                        cntvec = cntvec + jnp.where(lanes == e, jnp.sum(mi), 0)
                    posv = basev + within
                    pref[r, sl] = posv
                    run_v[...] = run_v[...] + cntvec
                    plsc.store_scatter(z_v, [posv], tokv)
            return z

        lax.fori_loop(0, _NR, pbody, 0)
        pltpu.sync_copy(p1_v, pos0_hbm)
        pltpu.sync_copy(p2_v, pos1_hbm)
        pltpu.sync_copy(z_v, slot_tok_hbm)

        ue_v[...] = apo_exc + tot
        for half in range(2):
            tb = (lax.iota(jnp.int32, 16) + half * 16) * M
            te = jnp.zeros((16,), jnp.int32)
            for e in range(E):
                ae = jnp.sum(jnp.where(lanes == e, apo_inc, 0))
                te = te + (tb >= ae).astype(jnp.int32)
            te = jnp.minimum(te, E - 1)
            ueg = plsc.load_gather(ue_v, [te])
            act = (tb < ueg).astype(jnp.int32)
            te_v[pl.ds(half * 16, 16)] = te
            act_v[pl.ds(half * 16, 16)] = act
        pltpu.sync_copy(te_v, te_hbm)
        pltpu.sync_copy(act_v, act_hbm)


def _routing(i1f, i2f, cnt):
    out_type = [
        jax.ShapeDtypeStruct((P,), jnp.int32),
        jax.ShapeDtypeStruct((_NR, 128), jnp.int32),
        jax.ShapeDtypeStruct((_NR, 128), jnp.int32),
        jax.ShapeDtypeStruct((32,), jnp.int32),
        jax.ShapeDtypeStruct((32,), jnp.int32),
    ]
    scratch = [
        pltpu.VMEM((_NR, 128), jnp.int32),
        pltpu.VMEM((_NR, 128), jnp.int32),
        pltpu.VMEM((_NR, 128), jnp.int32),
        pltpu.VMEM((_NR, 128), jnp.int32),
        pltpu.VMEM((P,), jnp.int32),
        pltpu.VMEM((16,), jnp.int32),
        pltpu.VMEM((16,), jnp.int32),
        pltpu.VMEM((32,), jnp.int32),
        pltpu.VMEM((32,), jnp.int32),
    ]
    fn = pl.kernel(_routing_body, out_type=out_type, mesh=_MESH,
                   compiler_params=_SC_PARAMS, scratch_types=scratch)
    return fn(i1f, i2f, cnt)


GCH = 48
NCHG = GR // GCH


def _gather_body(tok_hbm, h2_hbm, xs_hbm, idx_v, buf0, buf1, sem0, sem1):
    c = lax.axis_index("c")
    s = lax.axis_index("s")
    wid = s * NC + c
    base = wid * GR
    pltpu.sync_copy(tok_hbm.at[pl.ds(base, GR)], idx_v)
    bufs = (buf0, buf1)
    sems = (sem0, sem1)
    cps = []
    for k in range(NCHG):
        cps.append(pltpu.async_copy(
            h2_hbm.at[idx_v.at[pl.ds(k * GCH, GCH)]], bufs[k % 2],
            sems[k % 2]))
        if k >= 1:
            cps[k - 1].wait()
            pltpu.sync_copy(bufs[(k - 1) % 2],
                            xs_hbm.at[pl.ds(base + (k - 1) * GCH, GCH)])
    cps[NCHG - 1].wait()
    pltpu.sync_copy(bufs[(NCHG - 1) % 2],
                    xs_hbm.at[pl.ds(base + (NCHG - 1) * GCH, GCH)])


def _gather(slot_tok, h2):
    fn = pl.kernel(
        _gather_body,
        out_type=jax.ShapeDtypeStruct((P, D), jnp.float32),
        mesh=_MESH,
        compiler_params=_SC_PARAMS,
        scratch_types=[
            pltpu.VMEM((GR,), jnp.int32),
            pltpu.VMEM((GCH, D), jnp.float32),
            pltpu.VMEM((GCH, D), jnp.float32),
            pltpu.SemaphoreType.DMA,
            pltpu.SemaphoreType.DMA,
        ],
    )
    return fn(slot_tok, h2)


def _moe_body(te_ref, act_ref, xs_ref, w1_ref, w2_ref, ys_ref):
    i = pl.program_id(0)

    @pl.when(act_ref[i] == 1)
    def _():
        x = xs_ref[...]
        gu = _dot_t(x, w1_ref[0])
        g = gu[:, :FF]
        u = gu[:, FF:]
        a = g * jax.nn.sigmoid(g) * u
        ys_ref[...] = _dot_t(a, w2_ref[0])


def _moe_matmul(te, act, xs, W1, W2):
    grid_spec = pltpu.PrefetchScalarGridSpec(
        num_scalar_prefetch=2,
        grid=(NT,),
        in_specs=[
            pl.BlockSpec((M, D), lambda i, te, act: (i, 0)),
            pl.BlockSpec((1, 2 * FF, D), lambda i, te, act: (te[i], 0, 0)),
            pl.BlockSpec((1, D, FF), lambda i, te, act: (te[i], 0, 0)),
        ],
        out_specs=pl.BlockSpec((M, D), lambda i, te, act: (i, 0)),
    )
    return pl.pallas_call(
        _moe_body,
        grid_spec=grid_spec,
        out_shape=jax.ShapeDtypeStruct((P, D), jnp.float32),
    )(te, act, xs, W1, W2)


CCH = 32
NCHC = CT // CCH


def _combine_gather_body(pos0_hbm, pos1_hbm, ys_hbm, g0_hbm, g1_hbm,
                         i0_v, i1_v, b0_v, b1_v, sem0, sem1):
    c = lax.axis_index("c")
    s = lax.axis_index("s")
    wid = s * NC + c
    base = wid * CT
    pltpu.sync_copy(pos0_hbm.at[pl.ds(base, CT)], i0_v)
    pltpu.sync_copy(pos1_hbm.at[pl.ds(base, CT)], i1_v)
    bufs = (b0_v, b1_v)
    sems = (sem0, sem1)
    plan = []
    for k in range(NCHC):
        plan.append((i0_v, g0_hbm, k))
        plan.append((i1_v, g1_hbm, k))
    cps = []
    for j, (iv, dst, k) in enumerate(plan):
        cps.append(pltpu.async_copy(
            ys_hbm.at[iv.at[pl.ds(k * CCH, CCH)]], bufs[j % 2], sems[j % 2]))
        if j >= 1:
            piv, pdst, pk = plan[j - 1]
            cps[j - 1].wait()
            pltpu.sync_copy(bufs[(j - 1) % 2],
                            pdst.at[pl.ds(base + pk * CCH, CCH)])
    j = len(plan) - 1
    piv, pdst, pk = plan[j]
    cps[j].wait()
    pltpu.sync_copy(bufs[j % 2], pdst.at[pl.ds(base + pk * CCH, CCH)])


def _combine_gather(pos0, pos1, ys):
    fn = pl.kernel(
        _combine_gather_body,
        out_type=[jax.ShapeDtypeStruct((S, D), jnp.float32),
                  jax.ShapeDtypeStruct((S, D), jnp.float32)],
        mesh=_MESH,
        compiler_params=_SC_PARAMS,
        scratch_types=[
            pltpu.VMEM((CT,), jnp.int32),
            pltpu.VMEM((CT,), jnp.int32),
            pltpu.VMEM((CCH, D), jnp.float32),
            pltpu.VMEM((CCH, D), jnp.float32),
            pltpu.SemaphoreType.DMA,
            pltpu.SemaphoreType.DMA,
        ],
    )
    return fn(pos0, pos1, ys)


def _combine_add_body(g0_ref, g1_ref, w0_ref, w1_ref, out_ref):
    out_ref[...] = g0_ref[...] * w0_ref[...] + g1_ref[...] * w1_ref[...]


def _combine_add(g0, g1, w0, w1):
    rows = 512
    blk = pl.BlockSpec((rows, D), lambda i: (i, 0))
    cblk = pl.BlockSpec((rows, 1), lambda i: (i, 0))
    return pl.pallas_call(
        _combine_add_body,
        grid=(S // rows,),
        in_specs=[blk, blk, cblk, cblk],
        out_specs=blk,
        out_shape=jax.ShapeDtypeStruct((S, D), jnp.float32),
    )(g0, g1, w0, w1)


def kernel(hidden_states, residual, op_norm_w, ffn_norm_w, conv_in_W,
           conv_W, conv_out_W, gate_W, e_bias, W1, W2):
    B = hidden_states.shape[0]
    hs = hidden_states.reshape(S, D)
    res = residual.reshape(S, D)
    convt = conv_W.T
    res_out, h2, i1, i2, w0, w1, cnt = _pre_moe(
        hs, res, op_norm_w, ffn_norm_w, conv_in_W, convt, conv_out_W,
        gate_W, e_bias)
    slot_tok, pos0, pos1, te, act = _routing(
        i1.reshape(_NR, 128), i2.reshape(_NR, 128), cnt.reshape(16))
    xs = _gather(slot_tok, h2)
    ys = _moe_matmul(te, act, xs, W1, W2)
    g0, g1 = _combine_gather(pos0.reshape(S), pos1.reshape(S), ys)
    out = _combine_add(g0, g1, w0, w1)
    return out.reshape(B, S, D), res_out.reshape(B, S, D)


# speedup vs baseline: 1.3165x; 1.1313x over previous
import jax
import jax.numpy as jnp
from jax import lax
from jax.experimental import pallas as pl
from jax.experimental.pallas import tpu as pltpu
from jax.experimental.pallas import tpu_sc as plsc

EPS = 1e-05

S = 2048
D = 1024
E = 8
FF = 1024
ROWS_A = 256

NC, NS = 2, 16
M = 256
NT = 24
P = NT * M
TT = S // NS
GR = P // (NC * NS)
CT = S // (NC * NS)

_MESH = plsc.VectorSubcoreMesh(core_axis_name="c", subcore_axis_name="s",
                               num_cores=NC, num_subcores=NS)
_SC_PARAMS = pltpu.CompilerParams(needs_layout_passes=False)


def _rms(x, w):
    var = jnp.mean(x * x, axis=-1, keepdims=True)
    return x * jax.lax.rsqrt(var + EPS) * w


def _dot_t(a, b):
    return jax.lax.dot_general(a, b, (((1,), (1,)), ((), ())),
                               preferred_element_type=jnp.float32)


def _pre_moe_body(hs_ref, res_ref, hs_prev_ref, res_prev_ref, opw_ref,
                  ffw_ref, cin_ref, convt_ref, cout_ref, gw_ref, eb_ref,
                  res_out_ref, h2_ref, pack_ref, w0_ref, w1_ref,
                  cnt_ref, cacc_ref):
    i = pl.program_id(0)
    z = hs_ref[...] + res_ref[...]
    h = _rms(z, opw_ref[...])
    bcx = _dot_t(h, cin_ref[...])
    b = bcx[:, :D]
    c = bcx[:, D:2 * D]
    x = bcx[:, 2 * D:]
    bx = b * x
    zp = hs_prev_ref[ROWS_A - 2:, :] + res_prev_ref[ROWS_A - 2:, :]
    hp = _rms(zp, opw_ref[...])
    bxh = _dot_t(hp, cin_ref[:D, :]) * _dot_t(hp, cin_ref[2 * D:, :])
    bxh = jnp.where(i > 0, bxh, 0.0)
    bxp = jnp.concatenate([bxh, bx], axis=0)
    conv = (bxp[0:ROWS_A] * convt_ref[0:1, :]
            + bxp[1:ROWS_A + 1] * convt_ref[1:2, :]
            + bxp[2:ROWS_A + 2] * convt_ref[2:3, :])
    y = c * conv
    z2 = _dot_t(y, cout_ref[...]) + z
    res_out_ref[...] = z2
    h2 = _rms(z2, ffw_ref[...])
    h2_ref[...] = h2
    logits_t = jax.lax.dot_general(gw_ref[...], h2, (((1,), (1,)), ((), ())),
                                   preferred_element_type=jnp.float32)
    sc_t = jax.nn.sigmoid(logits_t)
    ch_t = sc_t + eb_ref[...]
    m1 = ch_t[0:1, :]
    s1 = sc_t[0:1, :]
    i1 = jnp.zeros((1, ROWS_A), jnp.int32)
    for k in range(1, E):
        ck = ch_t[k:k + 1, :]
        upd = ck > m1
        i1 = jnp.where(upd, k, i1)
        s1 = jnp.where(upd, sc_t[k:k + 1, :], s1)
        m1 = jnp.where(upd, ck, m1)
    m2 = jnp.full((1, ROWS_A), -jnp.inf, jnp.float32)
    s2 = jnp.zeros((1, ROWS_A), jnp.float32)
    i2 = jnp.zeros((1, ROWS_A), jnp.int32)
    for k in range(E):
        ck = ch_t[k:k + 1, :]
        upd = jnp.logical_and(i1 != k, ck > m2)
        i2 = jnp.where(upd, k, i2)
        s2 = jnp.where(upd, sc_t[k:k + 1, :], s2)
        m2 = jnp.where(upd, ck, m2)
    den = s1 + s2
    pack_ref[...] = jnp.concatenate(
        [i1, i2, jnp.zeros((6, ROWS_A), jnp.int32)], axis=0)
    w0_ref[...] = s1 / den
    w1_ref[...] = s2 / den
    part = jnp.zeros((1, 16), jnp.int32)
    for e in range(E):
        ce = (jnp.sum((i1 == e).astype(jnp.int32))
              + jnp.sum((i2 == e).astype(jnp.int32)))
        onehot = (lax.broadcasted_iota(jnp.int32, (1, 16), 1) == e)
        part = part + jnp.where(onehot, ce, 0)

    @pl.when(i == 0)
    def _():
        cacc_ref[...] = part

    @pl.when(i > 0)
    def _():
        cacc_ref[...] += part

    @pl.when(i == S // ROWS_A - 1)
    def _():
        cnt_ref[...] = cacc_ref[...]


def _pre_moe(hs, res, opw, ffw, cin, convt, cout, gw, eb):
    n = S // ROWS_A
    row_blk = pl.BlockSpec((ROWS_A, D), lambda i: (i, 0))
    prev_blk = pl.BlockSpec((ROWS_A, D), lambda i: (jnp.maximum(i - 1, 0), 0))
    const2 = lambda shape: pl.BlockSpec(shape, lambda i: (0, 0))
    out_shapes = [
        jax.ShapeDtypeStruct((S, D), jnp.float32),
        jax.ShapeDtypeStruct((S, D), jnp.float32),
        jax.ShapeDtypeStruct((8, S), jnp.int32),
        jax.ShapeDtypeStruct((1, S), jnp.float32),
        jax.ShapeDtypeStruct((1, S), jnp.float32),
        jax.ShapeDtypeStruct((1, 16), jnp.int32),
    ]
    return pl.pallas_call(
        _pre_moe_body,
        grid=(n,),
        in_specs=[row_blk, row_blk, prev_blk, prev_blk,
                  const2((1, D)), const2((1, D)), const2((3 * D, D)),
                  const2((3, D)), const2((D, D)), const2((E, D)),
                  const2((E, 1))],
        out_specs=[row_blk, row_blk,
                   pl.BlockSpec((8, ROWS_A), lambda i: (0, i)),
                   pl.BlockSpec((1, ROWS_A), lambda i: (0, i)),
                   pl.BlockSpec((1, ROWS_A), lambda i: (0, i)),
                   pl.BlockSpec((1, 16), lambda i: (0, 0))],
        out_shape=out_shapes,
        scratch_shapes=[pltpu.VMEM((1, 16), jnp.int32)],
    )(hs, res, hs, res, opw.reshape(1, D), ffw.reshape(1, D), cin, convt,
      cout, gw, eb.reshape(E, 1))


_NR = S // 128


def _routing_body(pack_hbm, cnt_hbm,
                  slot_tok_hbm, pos0_hbm, pos1_hbm,
                  te_hbm, act_hbm,
                  iv1_v, iv2_v, p1_v, p2_v, z_v,
                  run_v, ue_v, te_v, act_v):
    c = lax.axis_index("c")
    s = lax.axis_index("s")

    @pl.when(jnp.logical_and(c == 0, s == 0))
    def _():
        lanes = lax.iota(jnp.int32, 16)
        for j in range(0, P, 16):
            z_v[pl.ds(j, 16)] = jnp.full((16,), j & (S - 1), jnp.int32) + lanes
        pltpu.sync_copy(pack_hbm.at[0], iv1_v)
        pltpu.sync_copy(pack_hbm.at[1], iv2_v)
        pltpu.sync_copy(cnt_hbm.at[0], run_v)
        tot = run_v[...]
        padded = ((tot + (M - 1)) >> 8) << 8
        apo_inc = plsc.cumsum(padded)
        apo_exc = apo_inc - padded
        run_v[...] = apo_exc

        def pbody(r, z):
            for gg in range(128 // 16):
                sl = pl.ds(gg * 16, 16)
                tokv = r * 128 + gg * 16 + lanes
                for iref, pref in ((iv1_v, p1_v), (iv2_v, p2_v)):
                    vec = iref[pl.ds(r * 128 + gg * 16, 16)]
                    basev = plsc.load_gather(run_v, [vec])
                    within = jnp.zeros((16,), jnp.int32)
                    cntvec = jnp.zeros((16,), jnp.int32)
                    for e in range(E):
                        m = vec == e
                        mi = m.astype(jnp.int32)
                        cm = plsc.cumsum(mi)
                        within = jnp.where(m, cm - 1, within)
                        cntvec = cntvec + jnp.where(lanes == e, jnp.sum(mi), 0)
                    posv = basev + within
                    pref[r, sl] = posv
                    run_v[...] = run_v[...] + cntvec
                    plsc.store_scatter(z_v, [posv], tokv)
            return z

        lax.fori_loop(0, _NR, pbody, 0)
        pltpu.sync_copy(p1_v, pos0_hbm)
        pltpu.sync_copy(p2_v, pos1_hbm)
        pltpu.sync_copy(z_v, slot_tok_hbm)

        ue_v[...] = apo_exc + tot
        for half in range(2):
            tb = (lax.iota(jnp.int32, 16) + half * 16) * M
            te = jnp.zeros((16,), jnp.int32)
            for e in range(E):
                ae = jnp.sum(jnp.where(lanes == e, apo_inc, 0))
                te = te + (tb >= ae).astype(jnp.int32)
            te = jnp.minimum(te, E - 1)
            ueg = plsc.load_gather(ue_v, [te])
            act = (tb < ueg).astype(jnp.int32)
            te_v[pl.ds(half * 16, 16)] = te
            act_v[pl.ds(half * 16, 16)] = act
        pltpu.sync_copy(te_v, te_hbm)
        pltpu.sync_copy(act_v, act_hbm)


def _routing(pack, cnt):
    out_type = [
        jax.ShapeDtypeStruct((P,), jnp.int32),
        jax.ShapeDtypeStruct((_NR, 128), jnp.int32),
        jax.ShapeDtypeStruct((_NR, 128), jnp.int32),
        jax.ShapeDtypeStruct((32,), jnp.int32),
        jax.ShapeDtypeStruct((32,), jnp.int32),
    ]
    scratch = [
        pltpu.VMEM((S,), jnp.int32),
        pltpu.VMEM((S,), jnp.int32),
        pltpu.VMEM((_NR, 128), jnp.int32),
        pltpu.VMEM((_NR, 128), jnp.int32),
        pltpu.VMEM((P,), jnp.int32),
        pltpu.VMEM((16,), jnp.int32),
        pltpu.VMEM((16,), jnp.int32),
        pltpu.VMEM((32,), jnp.int32),
        pltpu.VMEM((32,), jnp.int32),
    ]
    fn = pl.kernel(_routing_body, out_type=out_type, mesh=_MESH,
                   compiler_params=_SC_PARAMS, scratch_types=scratch)
    return fn(pack, cnt)


GCH = 48
NCHG = GR // GCH


def _gather_body(tok_hbm, h2_hbm, xs_hbm, idx_v, buf0, buf1, sem0, sem1):
    c = lax.axis_index("c")
    s = lax.axis_index("s")
    wid = s * NC + c
    base = wid * GR
    pltpu.sync_copy(tok_hbm.at[pl.ds(base, GR)], idx_v)
    bufs = (buf0, buf1)
    sems = (sem0, sem1)
    cps = []
    for k in range(NCHG):
        cps.append(pltpu.async_copy(
            h2_hbm.at[idx_v.at[pl.ds(k * GCH, GCH)]], bufs[k % 2],
            sems[k % 2]))
        if k >= 1:
            cps[k - 1].wait()
            pltpu.sync_copy(bufs[(k - 1) % 2],
                            xs_hbm.at[pl.ds(base + (k - 1) * GCH, GCH)])
    cps[NCHG - 1].wait()
    pltpu.sync_copy(bufs[(NCHG - 1) % 2],
                    xs_hbm.at[pl.ds(base + (NCHG - 1) * GCH, GCH)])


def _gather(slot_tok, h2):
    fn = pl.kernel(
        _gather_body,
        out_type=jax.ShapeDtypeStruct((P, D), jnp.float32),
        mesh=_MESH,
        compiler_params=_SC_PARAMS,
        scratch_types=[
            pltpu.VMEM((GR,), jnp.int32),
            pltpu.VMEM((GCH, D), jnp.float32),
            pltpu.VMEM((GCH, D), jnp.float32),
            pltpu.SemaphoreType.DMA,
            pltpu.SemaphoreType.DMA,
        ],
    )
    return fn(slot_tok, h2)


def _moe_body(te_ref, act_ref, xs_ref, w1_ref, w2_ref, ys_ref):
    i = pl.program_id(0)

    @pl.when(act_ref[i] == 1)
    def _():
        x = xs_ref[...]
        gu = _dot_t(x, w1_ref[0])
        g = gu[:, :FF]
        u = gu[:, FF:]
        a = g * jax.nn.sigmoid(g) * u
        ys_ref[...] = _dot_t(a, w2_ref[0])


def _moe_matmul(te, act, xs, W1, W2):
    grid_spec = pltpu.PrefetchScalarGridSpec(
        num_scalar_prefetch=2,
        grid=(NT,),
        in_specs=[
            pl.BlockSpec((M, D), lambda i, te, act: (i, 0)),
            pl.BlockSpec((1, 2 * FF, D), lambda i, te, act: (te[i], 0, 0)),
            pl.BlockSpec((1, D, FF), lambda i, te, act: (te[i], 0, 0)),
        ],
        out_specs=pl.BlockSpec((M, D), lambda i, te, act: (i, 0)),
    )
    return pl.pallas_call(
        _moe_body,
        grid_spec=grid_spec,
        out_shape=jax.ShapeDtypeStruct((P, D), jnp.float32),
    )(te, act, xs, W1, W2)


CCH = 32
NCHC = CT // CCH


def _combine_body(pos0_hbm, pos1_hbm, w0_hbm, w1_hbm, ys_hbm, out_hbm,
                  i0_v, i1_v, wa_v, wb_v, b0_v, b1_v, ob_v, sem0, sem1):
    c = lax.axis_index("c")
    s = lax.axis_index("s")
    wid = s * NC + c
    base = wid * CT
    prow = wid // 2
    pcol = (wid % 2) * CT
    pltpu.sync_copy(pos0_hbm.at[prow, pl.ds(pcol, CT)], i0_v)
    pltpu.sync_copy(pos1_hbm.at[prow, pl.ds(pcol, CT)], i1_v)
    pltpu.sync_copy(w0_hbm.at[0, pl.ds(base, CT)], wa_v)
    pltpu.sync_copy(w1_hbm.at[0, pl.ds(base, CT)], wb_v)
    for chk in range(NCHC):
        t0 = base + chk * CCH
        cp0 = pltpu.async_copy(
            ys_hbm.at[i0_v.at[pl.ds(chk * CCH, CCH)]], b0_v, sem0)
        cp1 = pltpu.async_copy(
            ys_hbm.at[i1_v.at[pl.ds(chk * CCH, CCH)]], b1_v, sem1)
        cp0.wait()
        cp1.wait()

        def row_body(r, carry):
            w0s = plsc.load_gather(wa_v, [jnp.full((16,), chk * CCH, jnp.int32) + r])
            w1s = plsc.load_gather(wb_v, [jnp.full((16,), chk * CCH, jnp.int32) + r])
            for cc in range(D // 16):
                sl = pl.ds(cc * 16, 16)
                ob_v[r, sl] = b0_v[r, sl] * w0s + b1_v[r, sl] * w1s
            return carry

        lax.fori_loop(0, CCH, row_body, 0)
        pltpu.sync_copy(ob_v, out_hbm.at[pl.ds(t0, CCH)])


def _combine(pos0, pos1, w0, w1, ys):
    fn = pl.kernel(
        _combine_body,
        out_type=jax.ShapeDtypeStruct((S, D), jnp.float32),
        mesh=_MESH,
        compiler_params=_SC_PARAMS,
        scratch_types=[
            pltpu.VMEM((CT,), jnp.int32),
            pltpu.VMEM((CT,), jnp.int32),
            pltpu.VMEM((CT,), jnp.float32),
            pltpu.VMEM((CT,), jnp.float32),
            pltpu.VMEM((CCH, D), jnp.float32),
            pltpu.VMEM((CCH, D), jnp.float32),
            pltpu.VMEM((CCH, D), jnp.float32),
            pltpu.SemaphoreType.DMA,
            pltpu.SemaphoreType.DMA,
        ],
    )
    return fn(pos0, pos1, w0, w1, ys)


def kernel(hidden_states, residual, op_norm_w, ffn_norm_w, conv_in_W,
           conv_W, conv_out_W, gate_W, e_bias, W1, W2):
    B = hidden_states.shape[0]
    hs = hidden_states.reshape(S, D)
    res = residual.reshape(S, D)
    convt = conv_W.T
    res_out, h2, pack, w0, w1, cnt = _pre_moe(
        hs, res, op_norm_w, ffn_norm_w, conv_in_W, convt, conv_out_W,
        gate_W, e_bias)
    slot_tok, pos0, pos1, te, act = _routing(pack, cnt)
    xs = _gather(slot_tok, h2)
    ys = _moe_matmul(te, act, xs, W1, W2)
    out = _combine(pos0, pos1, w0, w1, ys)
    return out.reshape(B, S, D), res_out.reshape(B, S, D)


# speedup vs baseline: 1.3673x; 1.0386x over previous
import jax
import jax.numpy as jnp
from jax import lax
from jax.experimental import pallas as pl
from jax.experimental.pallas import tpu as pltpu
from jax.experimental.pallas import tpu_sc as plsc

EPS = 1e-05

S = 2048
D = 1024
E = 8
FF = 1024
ROWS_A = 512

NC, NS = 2, 16
M = 256
NT = 24
P = NT * M
TT = S // NS
GR = P // (NC * NS)
CT = S // (NC * NS)

_MESH = plsc.VectorSubcoreMesh(core_axis_name="c", subcore_axis_name="s",
                               num_cores=NC, num_subcores=NS)
_SC_PARAMS = pltpu.CompilerParams(needs_layout_passes=False)


def _rms(x, w):
    var = jnp.mean(x * x, axis=-1, keepdims=True)
    return x * jax.lax.rsqrt(var + EPS) * w


def _dot_t(a, b):
    return jax.lax.dot_general(a, b, (((1,), (1,)), ((), ())),
                               preferred_element_type=jnp.float32)


def _pre_moe_body(hs_ref, res_ref, hs_prev_ref, res_prev_ref, opw_ref,
                  ffw_ref, cin_ref, convt_ref, cout_ref, gw_ref, eb_ref,
                  res_out_ref, h2_ref, pack_ref, w0_ref, w1_ref,
                  cnt_ref, cacc_ref):
    i = pl.program_id(0)
    z = hs_ref[...] + res_ref[...]
    h = _rms(z, opw_ref[...])
    bcx = _dot_t(h, cin_ref[...])
    b = bcx[:, :D]
    c = bcx[:, D:2 * D]
    x = bcx[:, 2 * D:]
    bx = b * x
    zp = hs_prev_ref[ROWS_A - 2:, :] + res_prev_ref[ROWS_A - 2:, :]
    hp = _rms(zp, opw_ref[...])
    bxh = _dot_t(hp, cin_ref[:D, :]) * _dot_t(hp, cin_ref[2 * D:, :])
    bxh = jnp.where(i > 0, bxh, 0.0)
    bxp = jnp.concatenate([bxh, bx], axis=0)
    conv = (bxp[0:ROWS_A] * convt_ref[0:1, :]
            + bxp[1:ROWS_A + 1] * convt_ref[1:2, :]
            + bxp[2:ROWS_A + 2] * convt_ref[2:3, :])
    y = c * conv
    z2 = _dot_t(y, cout_ref[...]) + z
    res_out_ref[...] = z2
    h2 = _rms(z2, ffw_ref[...])
    h2_ref[...] = h2
    logits_t = jax.lax.dot_general(gw_ref[...], h2, (((1,), (1,)), ((), ())),
                                   preferred_element_type=jnp.float32)
    sc_t = jax.nn.sigmoid(logits_t)
    ch_t = sc_t + eb_ref[...]
    m1 = ch_t[0:1, :]
    s1 = sc_t[0:1, :]
    i1 = jnp.zeros((1, ROWS_A), jnp.int32)
    for k in range(1, E):
        ck = ch_t[k:k + 1, :]
        upd = ck > m1
        i1 = jnp.where(upd, k, i1)
        s1 = jnp.where(upd, sc_t[k:k + 1, :], s1)
        m1 = jnp.where(upd, ck, m1)
    m2 = jnp.full((1, ROWS_A), -jnp.inf, jnp.float32)
    s2 = jnp.zeros((1, ROWS_A), jnp.float32)
    i2 = jnp.zeros((1, ROWS_A), jnp.int32)
    for k in range(E):
        ck = ch_t[k:k + 1, :]
        upd = jnp.logical_and(i1 != k, ck > m2)
        i2 = jnp.where(upd, k, i2)
        s2 = jnp.where(upd, sc_t[k:k + 1, :], s2)
        m2 = jnp.where(upd, ck, m2)
    den = s1 + s2
    pack_ref[...] = jnp.concatenate(
        [i1, i2, jnp.zeros((6, ROWS_A), jnp.int32)], axis=0)
    w0_ref[...] = s1 / den
    w1_ref[...] = s2 / den
    part = jnp.zeros((1, 16), jnp.int32)
    for e in range(E):
        ce = (jnp.sum((i1 == e).astype(jnp.int32))
              + jnp.sum((i2 == e).astype(jnp.int32)))
        onehot = (lax.broadcasted_iota(jnp.int32, (1, 16), 1) == e)
        part = part + jnp.where(onehot, ce, 0)

    @pl.when(i == 0)
    def _():
        cacc_ref[...] = part

    @pl.when(i > 0)
    def _():
        cacc_ref[...] += part

    @pl.when(i == S // ROWS_A - 1)
    def _():
        cnt_ref[...] = cacc_ref[...]


def _pre_moe(hs, res, opw, ffw, cin, convt, cout, gw, eb):
    n = S // ROWS_A
    row_blk = pl.BlockSpec((ROWS_A, D), lambda i: (i, 0))
    prev_blk = pl.BlockSpec((ROWS_A, D), lambda i: (jnp.maximum(i - 1, 0), 0))
    const2 = lambda shape: pl.BlockSpec(shape, lambda i: (0, 0))
    out_shapes = [
        jax.ShapeDtypeStruct((S, D), jnp.float32),
        jax.ShapeDtypeStruct((S, D), jnp.float32),
        jax.ShapeDtypeStruct((8, S), jnp.int32),
        jax.ShapeDtypeStruct((1, S), jnp.float32),
        jax.ShapeDtypeStruct((1, S), jnp.float32),
        jax.ShapeDtypeStruct((1, 16), jnp.int32),
    ]
    return pl.pallas_call(
        _pre_moe_body,
        grid=(n,),
        in_specs=[row_blk, row_blk, prev_blk, prev_blk,
                  const2((1, D)), const2((1, D)), const2((3 * D, D)),
                  const2((3, D)), const2((D, D)), const2((E, D)),
                  const2((E, 1))],
        out_specs=[row_blk, row_blk,
                   pl.BlockSpec((8, ROWS_A), lambda i: (0, i)),
                   pl.BlockSpec((1, ROWS_A), lambda i: (0, i)),
                   pl.BlockSpec((1, ROWS_A), lambda i: (0, i)),
                   pl.BlockSpec((1, 16), lambda i: (0, 0))],
        out_shape=out_shapes,
        scratch_shapes=[pltpu.VMEM((1, 16), jnp.int32)],
    )(hs, res, hs, res, opw.reshape(1, D), ffw.reshape(1, D), cin, convt,
      cout, gw, eb.reshape(E, 1))


_NR = S // 128


def _routing_body(pack_hbm, cnt_hbm,
                  slot_tok_hbm, pos0_hbm, pos1_hbm,
                  te_hbm, act_hbm,
                  iv1_v, iv2_v, p1_v, p2_v, z_v,
                  run_v, ue_v, te_v, act_v):
    c = lax.axis_index("c")
    s = lax.axis_index("s")

    @pl.when(jnp.logical_and(c == 0, s == 0))
    def _():
        lanes = lax.iota(jnp.int32, 16)
        for j in range(0, P, 16):
            z_v[pl.ds(j, 16)] = jnp.full((16,), j & (S - 1), jnp.int32) + lanes
        pltpu.sync_copy(pack_hbm.at[0], iv1_v)
        pltpu.sync_copy(pack_hbm.at[1], iv2_v)
        pltpu.sync_copy(cnt_hbm.at[0], run_v)
        tot = run_v[...]
        padded = ((tot + (M - 1)) >> 8) << 8
        apo_inc = plsc.cumsum(padded)
        apo_exc = apo_inc - padded
        run_v[...] = apo_exc

        def pbody(r, z):
            for gg in range(128 // 16):
                sl = pl.ds(gg * 16, 16)
                tokv = r * 128 + gg * 16 + lanes
                for iref, pref in ((iv1_v, p1_v), (iv2_v, p2_v)):
                    vec = iref[pl.ds(r * 128 + gg * 16, 16)]
                    basev = plsc.load_gather(run_v, [vec])
                    within = jnp.zeros((16,), jnp.int32)
                    cntvec = jnp.zeros((16,), jnp.int32)
                    for e in range(E):
                        m = vec == e
                        mi = m.astype(jnp.int32)
                        cm = plsc.cumsum(mi)
                        within = jnp.where(m, cm - 1, within)
                        cntvec = cntvec + jnp.where(lanes == e, jnp.sum(mi), 0)
                    posv = basev + within
                    pref[r, sl] = posv
                    run_v[...] = run_v[...] + cntvec
                    plsc.store_scatter(z_v, [posv], tokv)
            return z

        lax.fori_loop(0, _NR, pbody, 0)
        pltpu.sync_copy(p1_v, pos0_hbm)
        pltpu.sync_copy(p2_v, pos1_hbm)
        pltpu.sync_copy(z_v, slot_tok_hbm)

        ue_v[...] = apo_exc + tot
        for half in range(2):
            tb = (lax.iota(jnp.int32, 16) + half * 16) * M
            te = jnp.zeros((16,), jnp.int32)
            for e in range(E):
                ae = jnp.sum(jnp.where(lanes == e, apo_inc, 0))
                te = te + (tb >= ae).astype(jnp.int32)
            te = jnp.minimum(te, E - 1)
            ueg = plsc.load_gather(ue_v, [te])
            act = (tb < ueg).astype(jnp.int32)
            te_v[pl.ds(half * 16, 16)] = te
            act_v[pl.ds(half * 16, 16)] = act
        pltpu.sync_copy(te_v, te_hbm)
        pltpu.sync_copy(act_v, act_hbm)


def _routing(pack, cnt):
    out_type = [
        jax.ShapeDtypeStruct((P,), jnp.int32),
        jax.ShapeDtypeStruct((_NR, 128), jnp.int32),
        jax.ShapeDtypeStruct((_NR, 128), jnp.int32),
        jax.ShapeDtypeStruct((32,), jnp.int32),
        jax.ShapeDtypeStruct((32,), jnp.int32),
    ]
    scratch = [
        pltpu.VMEM((S,), jnp.int32),
        pltpu.VMEM((S,), jnp.int32),
        pltpu.VMEM((_NR, 128), jnp.int32),
        pltpu.VMEM((_NR, 128), jnp.int32),
        pltpu.VMEM((P,), jnp.int32),
        pltpu.VMEM((16,), jnp.int32),
        pltpu.VMEM((16,), jnp.int32),
        pltpu.VMEM((32,), jnp.int32),
        pltpu.VMEM((32,), jnp.int32),
    ]
    fn = pl.kernel(_routing_body, out_type=out_type, mesh=_MESH,
                   compiler_params=_SC_PARAMS, scratch_types=scratch)
    return fn(pack, cnt)


GCH = 48
NCHG = GR // GCH


def _gather_body(tok_hbm, h2_hbm, xs_hbm, idx_v, buf0, buf1, sem0, sem1):
    c = lax.axis_index("c")
    s = lax.axis_index("s")
    wid = s * NC + c
    base = wid * GR
    pltpu.sync_copy(tok_hbm.at[pl.ds(base, GR)], idx_v)
    bufs = (buf0, buf1)
    sems = (sem0, sem1)
    cps = []
    for k in range(NCHG):
        cps.append(pltpu.async_copy(
            h2_hbm.at[idx_v.at[pl.ds(k * GCH, GCH)]], bufs[k % 2],
            sems[k % 2]))
        if k >= 1:
            cps[k - 1].wait()
            pltpu.sync_copy(bufs[(k - 1) % 2],
                            xs_hbm.at[pl.ds(base + (k - 1) * GCH, GCH)])
    cps[NCHG - 1].wait()
    pltpu.sync_copy(bufs[(NCHG - 1) % 2],
                    xs_hbm.at[pl.ds(base + (NCHG - 1) * GCH, GCH)])


def _gather(slot_tok, h2):
    fn = pl.kernel(
        _gather_body,
        out_type=jax.ShapeDtypeStruct((P, D), jnp.float32),
        mesh=_MESH,
        compiler_params=_SC_PARAMS,
        scratch_types=[
            pltpu.VMEM((GR,), jnp.int32),
            pltpu.VMEM((GCH, D), jnp.float32),
            pltpu.VMEM((GCH, D), jnp.float32),
            pltpu.SemaphoreType.DMA,
            pltpu.SemaphoreType.DMA,
        ],
    )
    return fn(slot_tok, h2)


def _moe_body(te_ref, act_ref, xs_ref, w1_ref, w2_ref, ys_ref):
    i = pl.program_id(0)

    @pl.when(act_ref[i] == 1)
    def _():
        x = xs_ref[...]
        gu = _dot_t(x, w1_ref[0])
        g = gu[:, :FF]
        u = gu[:, FF:]
        a = g * jax.nn.sigmoid(g) * u
        ys_ref[...] = _dot_t(a, w2_ref[0])


def _moe_matmul(te, act, xs, W1, W2):
    grid_spec = pltpu.PrefetchScalarGridSpec(
        num_scalar_prefetch=2,
        grid=(NT,),
        in_specs=[
            pl.BlockSpec((M, D), lambda i, te, act: (i, 0)),
            pl.BlockSpec((1, 2 * FF, D), lambda i, te, act: (te[i], 0, 0)),
            pl.BlockSpec((1, D, FF), lambda i, te, act: (te[i], 0, 0)),
        ],
        out_specs=pl.BlockSpec((M, D), lambda i, te, act: (i, 0)),
    )
    return pl.pallas_call(
        _moe_body,
        grid_spec=grid_spec,
        out_shape=jax.ShapeDtypeStruct((P, D), jnp.float32),
    )(te, act, xs, W1, W2)


CCH = 32
NCHC = CT // CCH


def _combine_body(pos0_hbm, pos1_hbm, w0_hbm, w1_hbm, ys_hbm, out_hbm,
                  i0_v, i1_v, wa_v, wb_v, b0_v, b1_v, ob_v, sem0, sem1):
    c = lax.axis_index("c")
    s = lax.axis_index("s")
    wid = s * NC + c
    base = wid * CT
    prow = wid // 2
    pcol = (wid % 2) * CT
    pltpu.sync_copy(pos0_hbm.at[prow, pl.ds(pcol, CT)], i0_v)
    pltpu.sync_copy(pos1_hbm.at[prow, pl.ds(pcol, CT)], i1_v)
    pltpu.sync_copy(w0_hbm.at[0, pl.ds(base, CT)], wa_v)
    pltpu.sync_copy(w1_hbm.at[0, pl.ds(base, CT)], wb_v)
    for chk in range(NCHC):
        t0 = base + chk * CCH
        cp0 = pltpu.async_copy(
            ys_hbm.at[i0_v.at[pl.ds(chk * CCH, CCH)]], b0_v, sem0)
        cp1 = pltpu.async_copy(
            ys_hbm.at[i1_v.at[pl.ds(chk * CCH, CCH)]], b1_v, sem1)
        cp0.wait()
        cp1.wait()

        def row_body(r, carry):
            w0s = plsc.load_gather(wa_v, [jnp.full((16,), chk * CCH, jnp.int32) + r])
            w1s = plsc.load_gather(wb_v, [jnp.full((16,), chk * CCH, jnp.int32) + r])
            for cc in range(D // 16):
                sl = pl.ds(cc * 16, 16)
                ob_v[r, sl] = b0_v[r, sl] * w0s + b1_v[r, sl] * w1s
            return carry

        lax.fori_loop(0, CCH, row_body, 0)
        pltpu.sync_copy(ob_v, out_hbm.at[pl.ds(t0, CCH)])


def _combine(pos0, pos1, w0, w1, ys):
    fn = pl.kernel(
        _combine_body,
        out_type=jax.ShapeDtypeStruct((S, D), jnp.float32),
        mesh=_MESH,
        compiler_params=_SC_PARAMS,
        scratch_types=[
            pltpu.VMEM((CT,), jnp.int32),
            pltpu.VMEM((CT,), jnp.int32),
            pltpu.VMEM((CT,), jnp.float32),
            pltpu.VMEM((CT,), jnp.float32),
            pltpu.VMEM((CCH, D), jnp.float32),
            pltpu.VMEM((CCH, D), jnp.float32),
            pltpu.VMEM((CCH, D), jnp.float32),
            pltpu.SemaphoreType.DMA,
            pltpu.SemaphoreType.DMA,
        ],
    )
    return fn(pos0, pos1, w0, w1, ys)


def kernel(hidden_states, residual, op_norm_w, ffn_norm_w, conv_in_W,
           conv_W, conv_out_W, gate_W, e_bias, W1, W2):
    B = hidden_states.shape[0]
    hs = hidden_states.reshape(S, D)
    res = residual.reshape(S, D)
    convt = conv_W.T
    res_out, h2, pack, w0, w1, cnt = _pre_moe(
        hs, res, op_norm_w, ffn_norm_w, conv_in_W, convt, conv_out_W,
        gate_W, e_bias)
    slot_tok, pos0, pos1, te, act = _routing(pack, cnt)
    xs = _gather(slot_tok, h2)
    ys = _moe_matmul(te, act, xs, W1, W2)
    out = _combine(pos0, pos1, w0, w1, ys)
    return out.reshape(B, S, D), res_out.reshape(B, S, D)
